# baseline (speedup 1.0000x reference)
import functools

import jax
import jax.numpy as jnp
from jax import lax
from jax.experimental import pallas as pl
from jax.experimental.pallas import tpu as pltpu
from jax.experimental.pallas import tpu_sc as plsc

N_NODES = 10000
N_EDGES = 320000
D = 128
N_LAYERS = 4
EPS_BN = 1e-5

NC, NS = 2, 16
NW = NC * NS
CH = 128
GSPLIT = 2
GCH = CH // GSPLIT
CPT = 80
E_PAD = NW * CPT * CH
N_ACC = N_NODES + CH
HALF = CPT // 2
RING = 2
R_CHUNK = 624
N_TRUNC = R_CHUNK * NS
R_TAIL = N_NODES - N_TRUNC


def _sc_segment_sum(h, src, dst, zinit):
    mesh = plsc.VectorSubcoreMesh(core_axis_name="c", subcore_axis_name="s")

    @functools.partial(
        pl.kernel,
        out_type=jax.ShapeDtypeStruct((NC, N_NODES, D), jnp.float32),
        mesh=mesh,
        scratch_types=[
            pltpu.VMEM_SHARED((N_ACC, D), jnp.float32),
            pltpu.VMEM((HALF, CH), jnp.int32),
            pltpu.VMEM((CPT, CH), jnp.int32),
            pltpu.VMEM((RING, CH, D), jnp.float32),
        ] + [pltpu.SemaphoreType.DMA] * (RING * GSPLIT),
    )
    def ksc(h_hbm, src_hbm, dst_hbm, zin_hbm, out_hbm,
            acc, sidx, didx, rows, *sems):
        c = lax.axis_index("c")
        s = lax.axis_index("s")
        r0 = s * R_CHUNK

        @pl.when(c == 0)
        def _():
            pltpu.sync_copy(h_hbm.at[pl.ds(r0, R_CHUNK)],
                            acc.at[pl.ds(r0, R_CHUNK)])

            @pl.when(s == 0)
            def _():
                pltpu.sync_copy(h_hbm.at[pl.ds(N_TRUNC, R_TAIL)],
                                acc.at[pl.ds(N_TRUNC, R_TAIL)])

        @pl.when(c != 0)
        def _():
            pltpu.sync_copy(zin_hbm.at[pl.ds(0, R_CHUNK)],
                            acc.at[pl.ds(r0, R_CHUNK)])

            @pl.when(s == 0)
            def _():
                pltpu.sync_copy(zin_hbm.at[pl.ds(0, R_TAIL)],
                                acc.at[pl.ds(N_TRUNC, R_TAIL)])

        plsc.subcore_barrier()

        row0 = (c * NS + s) * CPT

        pltpu.sync_copy(dst_hbm.at[pl.ds(row0, CPT)], didx)

        def fire(l, b):
            for p in range(GSPLIT):
                pltpu.async_copy(h_hbm.at[sidx.at[l, pl.ds(p * GCH, GCH)]],
                                 rows.at[b, pl.ds(p * GCH, GCH)],
                                 sems[b * GSPLIT + p])

        def wait(l, b):
            for p in range(GSPLIT):
                pltpu.make_async_copy(
                    h_hbm.at[sidx.at[l, pl.ds(p * GCH, GCH)]],
                    rows.at[b, pl.ds(p * GCH, GCH)],
                    sems[b * GSPLIT + p]).wait()

        for h in range(2):
            pltpu.sync_copy(src_hbm.at[pl.ds(row0 + h * HALF, HALF)], sidx)
            for b in range(RING):
                fire(b, b)

            @pl.loop(0, HALF, step=RING)
            def _(g):
                for b in range(RING):
                    l = g + b
                    wait(l, b)
                    pltpu.sync_copy(rows.at[b],
                                    acc.at[didx.at[h * HALF + l]], add=True)
                    nxt = l + RING

                    @pl.when(nxt < HALF)
                    def _():
                        fire(nxt, b)

        plsc.subcore_barrier()
        pltpu.sync_copy(acc.at[pl.ds(r0, R_CHUNK)],
                        out_hbm.at[c, pl.ds(r0, R_CHUNK)])

        @pl.when(s == 0)
        def _():
            pltpu.sync_copy(acc.at[pl.ds(N_TRUNC, R_TAIL)],
                            out_hbm.at[c, pl.ds(N_TRUNC, R_TAIL)])

    return ksc(h, src, dst, zinit)


def _tc_layer(p0, p1, W1, W2, g1, b1, g2, b2, pW, pb, score, last):
    if last:
        outs = jax.ShapeDtypeStruct((1, D), jnp.float32)
    else:
        outs = (jax.ShapeDtypeStruct((N_NODES, D), jnp.float32),
                jax.ShapeDtypeStruct((1, D), jnp.float32))

    def body(p0_r, p1_r, W1_r, W2_r, g1_r, b1_r, g2_r, b2_r, pW_r, pb_r,
             sc_r, *o):
        m = p0_r[...] + p1_r[...]
        y = jnp.dot(m, W1_r[...], preferred_element_type=jnp.float32)
        mu = jnp.mean(y, axis=0, keepdims=True)
        yc = y - mu
        var = jnp.mean(yc * yc, axis=0, keepdims=True)
        y = jnp.maximum(g1_r[...] * yc * lax.rsqrt(var + EPS_BN) + b1_r[...],
                        0.0)
        z = jnp.dot(y, W2_r[...], preferred_element_type=jnp.float32)
        mu2 = jnp.mean(z, axis=0, keepdims=True)
        zc = z - mu2
        var2 = jnp.mean(zc * zc, axis=0, keepdims=True)
        hn = jnp.maximum(g2_r[...] * zc * lax.rsqrt(var2 + EPS_BN) + b2_r[...],
                         0.0)
        pooled = jnp.sum(hn, axis=0, keepdims=True)
        snew = (sc_r[...]
                + jnp.dot(pooled, pW_r[...], preferred_element_type=jnp.float32)
                + pb_r[...])
        if last:
            o[0][...] = snew
        else:
            o[0][...] = hn
            o[1][...] = snew

    return pl.pallas_call(body, out_shape=outs)(
        p0, p1, W1, W2, g1, b1, g2, b2, pW, pb, score)


def kernel(x, edge_index, fc1_W, fc2_W, bn1_gamma, bn1_beta,
           bn2_gamma, bn2_beta, pred_W, pred_b):
    pad_n = E_PAD - N_EDGES
    spread = jnp.arange(pad_n, dtype=jnp.int32) % CH
    src = jnp.concatenate(
        [edge_index[0].astype(jnp.int32), spread]).reshape(E_PAD // CH, CH)
    dst = jnp.concatenate(
        [edge_index[1].astype(jnp.int32),
         N_NODES + spread]).reshape(E_PAD // CH, CH)
    zinit = jnp.zeros((R_CHUNK, D), jnp.float32)
    score = jnp.zeros((1, D), jnp.float32)
    h = x
    for l in range(N_LAYERS):
        parts = _sc_segment_sum(h, src, dst, zinit)
        args = (parts[0], parts[1], fc1_W[l], fc2_W[l],
                bn1_gamma[l].reshape(1, D), bn1_beta[l].reshape(1, D),
                bn2_gamma[l].reshape(1, D), bn2_beta[l].reshape(1, D),
                pred_W[l], pred_b[l].reshape(1, D), score)
        if l < N_LAYERS - 1:
            h, score = _tc_layer(*args, last=False)
        else:
            score = _tc_layer(*args, last=True)
    return score


# speedup vs baseline: 1.0422x; 1.0422x over previous
import functools

import jax
import jax.numpy as jnp
from jax import lax
from jax.experimental import pallas as pl
from jax.experimental.pallas import tpu as pltpu
from jax.experimental.pallas import tpu_sc as plsc

N_NODES = 10000
N_EDGES = 320000
D = 128
N_LAYERS = 4
EPS_BN = 1e-5

NC, NS = 2, 16
NW = NC * NS
CH = 128
CPT = 80
E_PAD = NW * CPT * CH
N_ACC = N_NODES + CH
HALF = CPT // 2
RING = 2
R_CHUNK = 624
N_TRUNC = R_CHUNK * NS
R_TAIL = N_NODES - N_TRUNC


def _sc_segment_sum(h, src, dst, zinit):
    mesh = plsc.VectorSubcoreMesh(core_axis_name="c", subcore_axis_name="s")

    @functools.partial(
        pl.kernel,
        out_type=jax.ShapeDtypeStruct((NC, N_NODES, D), jnp.float32),
        mesh=mesh,
        scratch_types=[
            pltpu.VMEM_SHARED((N_ACC, D), jnp.float32),
            pltpu.VMEM((HALF, CH), jnp.int32),
            pltpu.VMEM((CPT, CH), jnp.int32),
            pltpu.VMEM((RING, CH, D), jnp.float32),
        ] + [pltpu.SemaphoreType.DMA] * RING,
    )
    def ksc(h_hbm, src_hbm, dst_hbm, zin_hbm, out_hbm,
            acc, sidx, didx, rows, *sems):
        c = lax.axis_index("c")
        s = lax.axis_index("s")
        r0 = s * R_CHUNK

        @pl.when(c == 0)
        def _():
            pltpu.sync_copy(h_hbm.at[pl.ds(r0, R_CHUNK)],
                            acc.at[pl.ds(r0, R_CHUNK)])

            @pl.when(s == 0)
            def _():
                pltpu.sync_copy(h_hbm.at[pl.ds(N_TRUNC, R_TAIL)],
                                acc.at[pl.ds(N_TRUNC, R_TAIL)])

        @pl.when(c != 0)
        def _():
            pltpu.sync_copy(zin_hbm.at[pl.ds(0, R_CHUNK)],
                            acc.at[pl.ds(r0, R_CHUNK)])

            @pl.when(s == 0)
            def _():
                pltpu.sync_copy(zin_hbm.at[pl.ds(0, R_TAIL)],
                                acc.at[pl.ds(N_TRUNC, R_TAIL)])

        plsc.subcore_barrier()

        row0 = (c * NS + s) * CPT

        pltpu.sync_copy(dst_hbm.at[pl.ds(row0, CPT)], didx)

        def fire(l, b):
            pltpu.async_copy(h_hbm.at[sidx.at[l]], rows.at[b], sems[b])

        for h in range(2):
            pltpu.sync_copy(src_hbm.at[pl.ds(row0 + h * HALF, HALF)], sidx)
            for b in range(RING):
                fire(b, b)

            @pl.loop(0, HALF, step=RING)
            def _(g):
                for b in range(RING):
                    l = g + b
                    pltpu.make_async_copy(h_hbm.at[sidx.at[l]],
                                          rows.at[b], sems[b]).wait()
                    pltpu.sync_copy(rows.at[b],
                                    acc.at[didx.at[h * HALF + l]], add=True)
                    nxt = l + RING

                    @pl.when(nxt < HALF)
                    def _():
                        fire(nxt, b)

        plsc.subcore_barrier()
        pltpu.sync_copy(acc.at[pl.ds(r0, R_CHUNK)],
                        out_hbm.at[c, pl.ds(r0, R_CHUNK)])

        @pl.when(s == 0)
        def _():
            pltpu.sync_copy(acc.at[pl.ds(N_TRUNC, R_TAIL)],
                            out_hbm.at[c, pl.ds(N_TRUNC, R_TAIL)])

    return ksc(h, src, dst, zinit)


def _tc_layer(p0, p1, W1, W2, g1, b1, g2, b2, pW, pb, score, last):
    if last:
        outs = jax.ShapeDtypeStruct((1, D), jnp.float32)
    else:
        outs = (jax.ShapeDtypeStruct((N_NODES, D), jnp.float32),
                jax.ShapeDtypeStruct((1, D), jnp.float32))

    def body(p0_r, p1_r, W1_r, W2_r, g1_r, b1_r, g2_r, b2_r, pW_r, pb_r,
             sc_r, *o):
        inv_n = 1.0 / N_NODES

        def bn_relu(t, g_r, b_r):
            mu = jnp.sum(t, axis=0, keepdims=True) * inv_n
            ex2 = jnp.sum(t * t, axis=0, keepdims=True) * inv_n
            a = g_r[...] * lax.rsqrt(ex2 - mu * mu + EPS_BN)
            return jnp.maximum(t * a + (b_r[...] - mu * a), 0.0)

        m = p0_r[...] + p1_r[...]
        y = jnp.dot(m, W1_r[...], preferred_element_type=jnp.float32)
        y = bn_relu(y, g1_r, b1_r)
        z = jnp.dot(y, W2_r[...], preferred_element_type=jnp.float32)
        hn = bn_relu(z, g2_r, b2_r)
        pooled = jnp.sum(hn, axis=0, keepdims=True)
        snew = (sc_r[...]
                + jnp.dot(pooled, pW_r[...], preferred_element_type=jnp.float32)
                + pb_r[...])
        if last:
            o[0][...] = snew
        else:
            o[0][...] = hn
            o[1][...] = snew

    return pl.pallas_call(body, out_shape=outs)(
        p0, p1, W1, W2, g1, b1, g2, b2, pW, pb, score)


def kernel(x, edge_index, fc1_W, fc2_W, bn1_gamma, bn1_beta,
           bn2_gamma, bn2_beta, pred_W, pred_b):
    pad_n = E_PAD - N_EDGES
    spread = jnp.arange(pad_n, dtype=jnp.int32) % CH
    src = jnp.concatenate(
        [edge_index[0].astype(jnp.int32), spread]).reshape(E_PAD // CH, CH)
    dst = jnp.concatenate(
        [edge_index[1].astype(jnp.int32),
         N_NODES + spread]).reshape(E_PAD // CH, CH)
    zinit = jnp.zeros((R_CHUNK, D), jnp.float32)
    score = jnp.zeros((1, D), jnp.float32)
    h = x
    for l in range(N_LAYERS):
        parts = _sc_segment_sum(h, src, dst, zinit)
        args = (parts[0], parts[1], fc1_W[l], fc2_W[l],
                bn1_gamma[l].reshape(1, D), bn1_beta[l].reshape(1, D),
                bn2_gamma[l].reshape(1, D), bn2_beta[l].reshape(1, D),
                pred_W[l], pred_b[l].reshape(1, D), score)
        if l < N_LAYERS - 1:
            h, score = _tc_layer(*args, last=False)
        else:
            score = _tc_layer(*args, last=True)
    return score


# speedup vs baseline: 1.1087x; 1.0638x over previous
import functools

import jax
import jax.numpy as jnp
from jax import lax
from jax.experimental import pallas as pl
from jax.experimental.pallas import tpu as pltpu
from jax.experimental.pallas import tpu_sc as plsc

N_NODES = 10000
N_EDGES = 320000
D = 128
N_LAYERS = 4
EPS_BN = 1e-5

NC, NS = 2, 16
NW = NC * NS
CH = 128
CPT = 80
E_PAD = NW * CPT * CH
N_ACC = N_NODES + CH
HALF = CPT // 2
RING = 2
R_CHUNK = 624
N_TRUNC = R_CHUNK * NS
R_TAIL = N_NODES - N_TRUNC


def _sc_segment_sum(h, src, dst, zinit):
    mesh = plsc.VectorSubcoreMesh(core_axis_name="c", subcore_axis_name="s")

    @functools.partial(
        pl.kernel,
        out_type=(jax.ShapeDtypeStruct((N_NODES, D), jnp.float32),
                  jax.ShapeDtypeStruct((N_NODES, D), jnp.float32)),
        mesh=mesh,
        scratch_types=[
            pltpu.VMEM_SHARED((N_ACC, D), jnp.float32),
            pltpu.VMEM((HALF, CH), jnp.int32),
            pltpu.VMEM((CPT, CH), jnp.int32),
            pltpu.VMEM((RING, CH, D), jnp.float32),
        ] + [pltpu.SemaphoreType.DMA] * RING,
    )
    def ksc(h_hbm, src_hbm, dst_hbm, zin_hbm, out0_hbm, out1_hbm,
            acc, sidx, didx, rows, *sems):
        c = lax.axis_index("c")
        s = lax.axis_index("s")
        r0 = s * R_CHUNK
        row0 = (c * NS + s) * CPT

        def fire(l, b):
            pltpu.async_copy(h_hbm.at[sidx.at[l]], rows.at[b], sems[b])

        pltpu.sync_copy(src_hbm.at[pl.ds(row0, HALF)], sidx)
        for b in range(RING):
            fire(b, b)

        pltpu.sync_copy(dst_hbm.at[pl.ds(row0, CPT)], didx)

        @pl.when(c == 0)
        def _():
            pltpu.sync_copy(h_hbm.at[pl.ds(r0, R_CHUNK)],
                            acc.at[pl.ds(r0, R_CHUNK)])

            @pl.when(s == 0)
            def _():
                pltpu.sync_copy(h_hbm.at[pl.ds(N_TRUNC, R_TAIL)],
                                acc.at[pl.ds(N_TRUNC, R_TAIL)])

        @pl.when(c != 0)
        def _():
            pltpu.sync_copy(zin_hbm.at[pl.ds(0, R_CHUNK)],
                            acc.at[pl.ds(r0, R_CHUNK)])

            @pl.when(s == 0)
            def _():
                pltpu.sync_copy(zin_hbm.at[pl.ds(0, R_TAIL)],
                                acc.at[pl.ds(N_TRUNC, R_TAIL)])

        plsc.subcore_barrier()

        for h in range(2):
            if h > 0:
                pltpu.sync_copy(src_hbm.at[pl.ds(row0 + h * HALF, HALF)],
                                sidx)
                for b in range(RING):
                    fire(b, b)

            @pl.loop(0, HALF, step=RING)
            def _(g):
                for b in range(RING):
                    l = g + b
                    pltpu.make_async_copy(h_hbm.at[sidx.at[l]],
                                          rows.at[b], sems[b]).wait()
                    pltpu.sync_copy(rows.at[b],
                                    acc.at[didx.at[h * HALF + l]], add=True)
                    nxt = l + RING

                    @pl.when(nxt < HALF)
                    def _():
                        fire(nxt, b)

        plsc.subcore_barrier()

        @pl.when(c == 0)
        def _():
            pltpu.sync_copy(acc.at[pl.ds(r0, R_CHUNK)],
                            out0_hbm.at[pl.ds(r0, R_CHUNK)])

            @pl.when(s == 0)
            def _():
                pltpu.sync_copy(acc.at[pl.ds(N_TRUNC, R_TAIL)],
                                out0_hbm.at[pl.ds(N_TRUNC, R_TAIL)])

        @pl.when(c != 0)
        def _():
            pltpu.sync_copy(acc.at[pl.ds(r0, R_CHUNK)],
                            out1_hbm.at[pl.ds(r0, R_CHUNK)])

            @pl.when(s == 0)
            def _():
                pltpu.sync_copy(acc.at[pl.ds(N_TRUNC, R_TAIL)],
                                out1_hbm.at[pl.ds(N_TRUNC, R_TAIL)])

    return ksc(h, src, dst, zinit)


def _tc_layer(p0, p1, W1, W2, g1, b1, g2, b2, pW, pb, score, last):
    if last:
        outs = jax.ShapeDtypeStruct((1, D), jnp.float32)
    else:
        outs = (jax.ShapeDtypeStruct((N_NODES, D), jnp.float32),
                jax.ShapeDtypeStruct((1, D), jnp.float32))

    def body(p0_r, p1_r, W1_r, W2_r, g1_r, b1_r, g2_r, b2_r, pW_r, pb_r,
             sc_r, *o):
        inv_n = 1.0 / N_NODES

        def bn_relu(t, g_r, b_r):
            mu = jnp.sum(t, axis=0, keepdims=True) * inv_n
            ex2 = jnp.sum(t * t, axis=0, keepdims=True) * inv_n
            a = g_r[...] * lax.rsqrt(ex2 - mu * mu + EPS_BN)
            return jnp.maximum(t * a + (b_r[...] - mu * a), 0.0)

        m = p0_r[...] + p1_r[...]
        y = jnp.dot(m, W1_r[...], preferred_element_type=jnp.float32)
        y = bn_relu(y, g1_r, b1_r)
        z = jnp.dot(y, W2_r[...], preferred_element_type=jnp.float32)
        hn = bn_relu(z, g2_r, b2_r)
        pooled = jnp.sum(hn, axis=0, keepdims=True)
        snew = (sc_r[...]
                + jnp.dot(pooled, pW_r[...], preferred_element_type=jnp.float32)
                + pb_r[...])
        if last:
            o[0][...] = snew
        else:
            o[0][...] = hn
            o[1][...] = snew

    return pl.pallas_call(body, out_shape=outs)(
        p0, p1, W1, W2, g1, b1, g2, b2, pW, pb, score)


def kernel(x, edge_index, fc1_W, fc2_W, bn1_gamma, bn1_beta,
           bn2_gamma, bn2_beta, pred_W, pred_b):
    pad_n = E_PAD - N_EDGES
    spread = jnp.arange(pad_n, dtype=jnp.int32) % CH
    src = jnp.concatenate(
        [edge_index[0].astype(jnp.int32), spread]).reshape(E_PAD // CH, CH)
    dst = jnp.concatenate(
        [edge_index[1].astype(jnp.int32),
         N_NODES + spread]).reshape(E_PAD // CH, CH)
    zinit = jnp.zeros((R_CHUNK, D), jnp.float32)
    score = jnp.zeros((1, D), jnp.float32)
    h = x
    for l in range(N_LAYERS):
        p0, p1 = _sc_segment_sum(h, src, dst, zinit)
        args = (p0, p1, fc1_W[l], fc2_W[l],
                bn1_gamma[l].reshape(1, D), bn1_beta[l].reshape(1, D),
                bn2_gamma[l].reshape(1, D), bn2_beta[l].reshape(1, D),
                pred_W[l], pred_b[l].reshape(1, D), score)
        if l < N_LAYERS - 1:
            h, score = _tc_layer(*args, last=False)
        else:
            score = _tc_layer(*args, last=True)
    return score
